# Initial kernel scaffold; baseline (speedup 1.0000x reference)
import jax
import jax.numpy as jnp
from jax.experimental import pallas as pl


def kernel(seeds, e_input, p_input, ep_adj, pe_adj, e2p_params, p2e_params):
    raise NotImplementedError("write your pallas kernel here")


# speedup vs baseline: 6.5895x; 6.5895x over previous
import functools

import jax
import jax.numpy as jnp
from jax import lax
from jax.experimental import pallas as pl
from jax.experimental.pallas import tpu as pltpu
from jax.experimental.pallas import tpu_sc as plsc

D_K = 50
P = 10000
E = 10000
NNZ = 320000
ALPHA = 0.5
TEMP = float(D_K) ** 0.5

NC = 2
NS = 16
NW = NC * NS
PER_W = NNZ // NW
CHUNK = 400
NCHUNK = PER_W // CHUNK
SROW = 80
HALF = P // 2
HROW = HALF + 8

_mesh = plsc.VectorSubcoreMesh(
    core_axis_name="c", subcore_axis_name="s", num_cores=NC, num_subcores=NS)
_sc_params = pltpu.CompilerParams(needs_layout_passes=False)

f32 = jnp.float32
i32 = jnp.int32


def _sds(shape):
    return jax.ShapeDtypeStruct(shape, f32)


def _tc_a_body(q_ref, kv_ref, seeds_ref,
               wls_ref, bls_ref, wks_ref, bks_ref, wvs_ref, bvs_ref,
               wres_ref, bres_ref, wgs_ref, bgs_ref,
               lq_ref, res_ref, kk_ref, vv_ref):
    inv_t = 1.0 / TEMP
    q = q_ref[...]
    kv = kv_ref[...]
    lq_ref[...] = (jnp.dot(q, wls_ref[...].T) + bls_ref[...]) * inv_t
    res_ref[...] = jnp.dot(q, wres_ref[...].T) + bres_ref[...]
    kk = jnp.dot(kv, wks_ref[...].T) + bks_ref[...]
    vv_ref[...] = jnp.dot(kv, wvs_ref[...].T) + bvs_ref[...]
    gq = (jnp.dot(seeds_ref[...], wgs_ref[...].T) + bgs_ref[...]) * inv_t
    eg = jnp.exp(jnp.sum(kk * gq, axis=1))
    lane = lax.broadcasted_iota(i32, kk.shape, 1)
    kk_ref[...] = jnp.where(lane == D_K, eg[:, None], kk)


def _tc_a(q, kv, seeds_flat, wls, bls, wks, bks, wvs, bvs, wres, bres,
          wgs, bgs, blk=1000):
    g = P // blk
    full2 = lambda a: pl.BlockSpec(a.shape, lambda i: (0,) * a.ndim)
    row2 = pl.BlockSpec((blk, 128), lambda i: (i, 0))
    args = (q, kv, seeds_flat, wls, bls, wks, bks, wvs, bvs, wres, bres,
            wgs, bgs)
    return pl.pallas_call(
        _tc_a_body,
        grid=(g,),
        in_specs=[row2, row2] + [full2(a) for a in args[2:]],
        out_specs=[row2] * 4,
        out_shape=[_sds((P, 128)), _sds((P, 128)), _sds((E, 128)),
                   _sds((E, 128))],
    )(*args)


def _flat2(r16):
    return [r16 >> 7, r16 & 127]


def _g1_body(row_hbm, col_hbm, lq_hbm, kk_hbm,
             lval_hbm, gval_hbm, ls_hbm, gs_hbm,
             ridx, cidx, lqbuf, kkbuf, lvbuf, gvbuf, ibuf, lacc, gacc,
             ls_sh, gs_sh, sem_a, sem_b):
    cid = lax.axis_index("c")
    sid = lax.axis_index("s")
    wid = sid * NC + cid
    base = wid * PER_W

    def z16(i, _):
        z = jnp.zeros((16,), f32)
        r = i >> 3
        c16 = (i & 7) * 16
        lacc[r, pl.ds(c16, 16)] = z
        gacc[r, pl.ds(c16, 16)] = z
        return None

    lax.fori_loop(0, SROW * 8, z16, None)

    def zib(i, _):
        ibuf[pl.ds(i * 16, 16)] = lax.iota(i32, 16) + i * 16
        return None

    lax.fori_loop(0, SROW // 16, zib, None)

    @pl.when(sid == 0)
    def _():
        pltpu.sync_copy(lacc, ls_sh)
        pltpu.sync_copy(gacc, gs_sh)

    plsc.subcore_barrier()

    def chunk(j, _):
        b = base + j * CHUNK
        pltpu.sync_copy(row_hbm.at[pl.ds(b, CHUNK)], ridx)
        pltpu.sync_copy(col_hbm.at[pl.ds(b, CHUNK)], cidx)
        cp_a = pltpu.async_copy(lq_hbm.at[ridx], lqbuf, sem_a)
        cp_b = pltpu.async_copy(kk_hbm.at[cidx], kkbuf, sem_b)
        cp_a.wait()
        cp_b.wait()

        def grp(i, _):
            e16 = lax.iota(i32, 16) + i * 16
            acc = jnp.zeros((16,), f32)
            for d in range(D_K):
                d16 = jnp.full((16,), d, i32)
                va = plsc.load_gather(lqbuf, [e16, d16])
                vb = plsc.load_gather(kkbuf, [e16, d16])
                acc = acc + va * vb
            lv16 = jnp.exp(acc)
            gv16 = plsc.load_gather(kkbuf, [e16, jnp.full((16,), D_K, i32)])
            r16 = ridx[pl.ds(i * 16, 16)]
            plsc.addupdate_scatter(lacc, _flat2(r16), lv16)
            plsc.addupdate_scatter(gacc, _flat2(r16), gv16)
            lvbuf[pl.ds(i * 16, 16)] = lv16
            gvbuf[pl.ds(i * 16, 16)] = gv16
            return None

        lax.fori_loop(0, CHUNK // 16, grp, None)
        pltpu.sync_copy(lvbuf, lval_hbm.at[pl.ds(b, CHUNK)])
        pltpu.sync_copy(gvbuf, gval_hbm.at[pl.ds(b, CHUNK)])
        return None

    lax.fori_loop(0, NCHUNK, chunk, None)

    pltpu.sync_copy(lacc, ls_sh.at[ibuf], add=True)
    pltpu.sync_copy(gacc, gs_sh.at[ibuf], add=True)
    plsc.subcore_barrier()

    @pl.when(sid == 0)
    def _():
        pltpu.sync_copy(ls_sh, ls_hbm.at[cid])
        pltpu.sync_copy(gs_sh, gs_hbm.at[cid])


def _sc_g1(row, col, lq128, kkx):
    return pl.kernel(
        _g1_body,
        out_type=(_sds((NNZ,)), _sds((NNZ,)), _sds((NC, SROW, 128)),
                  _sds((NC, SROW, 128))),
        mesh=_mesh,
        scratch_types=[
            pltpu.VMEM((CHUNK,), i32),
            pltpu.VMEM((CHUNK,), i32),
            pltpu.VMEM((CHUNK, 128), f32),
            pltpu.VMEM((CHUNK, 128), f32),
            pltpu.VMEM((CHUNK,), f32),
            pltpu.VMEM((CHUNK,), f32),
            pltpu.VMEM((SROW,), i32),
            pltpu.VMEM((SROW, 128), f32),
            pltpu.VMEM((SROW, 128), f32),
            pltpu.VMEM_SHARED((SROW, 128), f32),
            pltpu.VMEM_SHARED((SROW, 128), f32),
            pltpu.SemaphoreType.DMA,
            pltpu.SemaphoreType.DMA,
        ],
        compiler_params=_sc_params,
    )(row, col, lq128, kkx)


def _tc_c_body(ls_ref, gs_ref, il_ref, ig_ref):
    lsum = ls_ref[0] + ls_ref[1]
    gsum = gs_ref[0] + gs_ref[1]
    il_ref[...] = jnp.where(lsum > 0, (1.0 - ALPHA) / lsum, 0.0)
    ig_ref[...] = jnp.where(gsum > 0, ALPHA / gsum, 0.0)


def _tc_c(lsp, gsp):
    full = pl.BlockSpec((NC, SROW, 128), lambda: (0, 0, 0))
    out = pl.BlockSpec((SROW, 128), lambda: (0, 0))
    return pl.pallas_call(
        _tc_c_body,
        in_specs=[full, full],
        out_specs=[out, out],
        out_shape=[_sds((SROW, 128)), _sds((SROW, 128))],
    )(lsp, gsp)


def _g3_body(half, row_hbm, col_hbm, vv_hbm, lv_hbm, gv_hbm, il_hbm, ig_hbm,
             out_hbm, ridx, cidx, buf, lvbuf, gvbuf, cbuf, iltab, igtab,
             acc, sem):
    cid = lax.axis_index("c")
    sid = lax.axis_index("s")
    wid = sid * NC + cid
    base = wid * PER_W
    lo = half * HALF

    pltpu.sync_copy(il_hbm, iltab)
    pltpu.sync_copy(ig_hbm, igtab)

    def z16(i, _):
        buf[i >> 3, pl.ds((i & 7) * 16, 16)] = jnp.zeros((16,), f32)
        return None

    lax.fori_loop(0, 125 * 8, z16, None)
    zbase = sid * (HROW // NS)
    for off, ln in ((0, 125), (125, 125), (250, 63)):
        pltpu.sync_copy(buf.at[pl.ds(0, ln)],
                        acc.at[pl.ds(zbase + off, ln)])
    plsc.subcore_barrier()

    def chunk(j, _):
        b = base + j * CHUNK
        pltpu.sync_copy(row_hbm.at[pl.ds(b, CHUNK)], ridx)
        pltpu.sync_copy(col_hbm.at[pl.ds(b, CHUNK)], cidx)
        cp = pltpu.async_copy(vv_hbm.at[cidx], buf, sem)
        pltpu.sync_copy(lv_hbm.at[pl.ds(b, CHUNK)], lvbuf)
        pltpu.sync_copy(gv_hbm.at[pl.ds(b, CHUNK)], gvbuf)

        def coef(i, _):
            sl = pl.ds(i * 16, 16)
            r16 = ridx[sl]
            il16 = plsc.load_gather(iltab, _flat2(r16))
            ig16 = plsc.load_gather(igtab, _flat2(r16))
            c16 = ig16 * gvbuf[sl] + il16 * lvbuf[sl]
            rloc = r16 - lo
            valid = (rloc >= 0) & (rloc < HALF)
            ridx[sl] = jnp.where(valid, rloc, HALF)
            cbuf[sl] = jnp.where(valid, c16, 0.0)
            return None

        lax.fori_loop(0, CHUNK // 16, coef, None)
        cp.wait()

        def grp(i, _):
            c16v = cbuf[pl.ds(i * 16, 16)]
            for j16 in range(16):
                e = i * 16 + j16
                cv = c16v[j16]
                for k in range(8):
                    sl = pl.ds(k * 16, 16)
                    buf[e, sl] = buf[e, sl] * cv
            return None

        lax.fori_loop(0, CHUNK // 16, grp, None)
        pltpu.sync_copy(buf, acc.at[ridx], add=True)
        return None

    lax.fori_loop(0, NCHUNK, chunk, None)
    plsc.subcore_barrier()

    @pl.when(sid == 0)
    def _():
        pltpu.sync_copy(acc, out_hbm.at[cid])


def _sc_g3(row, col, vv, lval, gval, invl, invg, half):
    return pl.kernel(
        functools.partial(_g3_body, half),
        out_type=_sds((NC, HROW, 128)),
        mesh=_mesh,
        scratch_types=[
            pltpu.VMEM((CHUNK,), i32),
            pltpu.VMEM((CHUNK,), i32),
            pltpu.VMEM((CHUNK, 128), f32),
            pltpu.VMEM((CHUNK,), f32),
            pltpu.VMEM((CHUNK,), f32),
            pltpu.VMEM((CHUNK,), f32),
            pltpu.VMEM((SROW, 128), f32),
            pltpu.VMEM((SROW, 128), f32),
            pltpu.VMEM_SHARED((HROW, 128), f32),
            pltpu.SemaphoreType.DMA,
        ],
        compiler_params=_sc_params,
    )(row, col, vv, lval, gval, invl, invg)


def _tc_d_body(a0_ref, a1_ref, res_ref, g_ref, b_ref, out_ref):
    pid = pl.program_id(0)
    a = jnp.where(pid * a0_ref.shape[1] < HALF, a0_ref[...], a1_ref[...])
    pre = a[0] + a[1]
    h = jnp.maximum(pre + res_ref[...], 0.0)
    mu = jnp.mean(h, axis=1, keepdims=True)
    var = jnp.mean((h - mu) ** 2, axis=1, keepdims=True)
    out_ref[...] = (h - mu) / jnp.sqrt(var + 1e-5) * g_ref[...] + b_ref[...]


def _tc_d(a0, a1, res, ln_g, ln_b, blk=1000):
    g = P // blk
    nh = HALF // blk
    return pl.pallas_call(
        _tc_d_body,
        grid=(g,),
        in_specs=[
            pl.BlockSpec((NC, blk, 128),
                         lambda i: (0, jnp.minimum(i, nh - 1), 0)),
            pl.BlockSpec((NC, blk, 128),
                         lambda i: (0, jnp.clip(i - nh, 0, nh - 1), 0)),
            pl.BlockSpec((blk, 128), lambda i: (i, 0)),
            pl.BlockSpec((1, 128), lambda i: (0, 0)),
            pl.BlockSpec((1, 128), lambda i: (0, 0))],
        out_specs=pl.BlockSpec((blk, 128), lambda i: (i, 0)),
        out_shape=_sds((P, 128)),
    )(a0, a1, res, ln_g, ln_b)


def _pad_w(w, b):
    wp = jnp.zeros((128, w.shape[1]), f32).at[:D_K].set(w)
    bp = jnp.zeros((1, 128), f32).at[0, :D_K].set(b)
    return wp, bp


def _layer(p, seeds_flat, local_q, k_in, v_in, adj):
    row = adj[0].astype(i32)
    col = adj[1].astype(i32)
    wls, bls = _pad_w(p['wls_W'], p['wls_b'])
    wks, bks = _pad_w(p['wks_W'], p['wks_b'])
    wgs, bgs = _pad_w(p['wgs_W'], p['wgs_b'])
    bvs = p['wvs_b'].reshape(1, -1)
    bres = p['wres_b'].reshape(1, -1)

    lq128, res, kkx, vv = _tc_a(
        local_q, k_in, seeds_flat, wls, bls, wks, bks, p['wvs_W'], bvs,
        p['wres_W'], bres, wgs, bgs)
    lval, gval, lsp, gsp = _sc_g1(row, col, lq128, kkx)
    invl, invg = _tc_c(lsp, gsp)
    a0 = _sc_g3(row, col, vv, lval, gval, invl, invg, half=0)
    a1 = _sc_g3(row, col, vv, lval, gval, invl, invg, half=1)
    return _tc_d(a0, a1, res, p['ln_g'].reshape(1, -1),
                 p['ln_b'].reshape(1, -1))


def kernel(seeds, e_input, p_input, ep_adj, pe_adj, e2p_params, p2e_params):
    seeds_flat = seeds.reshape(1, -1)
    p_output = _layer(e2p_params, seeds_flat, p_input, e_input, e_input,
                      pe_adj)
    e_output = _layer(p2e_params, seeds_flat, e_input, p_output, p_output,
                      ep_adj)
    return (e_output, p_output)
